# Initial kernel scaffold; baseline (speedup 1.0000x reference)
import jax
import jax.numpy as jnp
from jax.experimental import pallas as pl


def kernel(x, edge_index, edge_attr, masked_tokens, batch, enc_w, prelu_w, mask_embed, e1_0, e2_0, w1_0, b1_0, w2_0, b2_0, g_0, be_0, e1_1, e2_1, w1_1, b1_1, w2_1, b2_1, g_1, be_1):
    raise NotImplementedError("write your pallas kernel here")


# speedup vs baseline: 2.5812x; 2.5812x over previous
import numpy as np
import jax
import jax.numpy as jnp
from jax import lax
from jax.experimental import pallas as pl
from jax.experimental.pallas import tpu as pltpu
from jax.experimental.pallas import tpu_sc as plsc

N = 10000
E = 160000
D = 256
H = 128
NT = 16
EPT = E // NT
K = 80
NCH = EPT // K
NP = 10240
RPT = NP // NT
ZR = 8
BR = 400
EPT2 = E // (2 * NT)
K2 = 40
NCH2 = EPT2 // K2
EPS = 1e-5


def _spmm_body(args):
    (hlo, hhi, src_h, dst_h,
     agglo, agghi,
     src_v, dst_v, buf, zbuf, agg_sh, gsem) = args

    c = lax.axis_index("c")
    s = lax.axis_index("s")
    zero16 = jnp.zeros((16,), jnp.float32)

    def zrow(r, _):
        def zcol(j, _):
            zbuf[r, pl.ds(j * 16, 16)] = zero16
            return 0
        return lax.fori_loop(0, H // 16, zcol, 0)
    lax.fori_loop(0, ZR, zrow, 0)

    def zcopy(k, _):
        pltpu.sync_copy(zbuf, agg_sh.at[pl.ds(s * RPT + k * ZR, ZR)])
        return 0
    lax.fori_loop(0, RPT // ZR, zcopy, 0)

    pltpu.sync_copy(src_h.at[s], src_v)
    pltpu.sync_copy(dst_h.at[s], dst_v)

    plsc.subcore_barrier()

    def chunk(i, _):
        isl = pl.ds(i * K, K)

        @pl.when(c == 0)
        def _():
            pltpu.async_copy(hlo.at[src_v.at[isl]], buf, gsem).wait()

        @pl.when(c == 1)
        def _():
            pltpu.async_copy(hhi.at[src_v.at[isl]], buf, gsem).wait()

        pltpu.sync_copy(buf, agg_sh.at[dst_v.at[i]], add=True)
        return 0
    lax.fori_loop(0, NCH, chunk, 0)

    plsc.subcore_barrier()

    wsl = pl.ds(s * RPT, RPT)

    @pl.when(c == 0)
    def _():
        pltpu.sync_copy(agg_sh.at[wsl], agglo.at[wsl])

    @pl.when(c == 1)
    def _():
        pltpu.sync_copy(agg_sh.at[wsl], agghi.at[wsl])


def _make_spmm():
    mesh = plsc.VectorSubcoreMesh(core_axis_name="c", subcore_axis_name="s")
    out_type = [jax.ShapeDtypeStruct((NP, H), jnp.float32),
                jax.ShapeDtypeStruct((NP, H), jnp.float32)]
    scratch = [
        pltpu.VMEM((EPT,), jnp.int32),
        pltpu.VMEM((NCH, K), jnp.int32),
        pltpu.VMEM((K, H), jnp.float32),
        pltpu.VMEM((ZR, H), jnp.float32),
        pltpu.VMEM_SHARED((NP, H), jnp.float32),
        pltpu.SemaphoreType.DMA,
    ]

    def body(*args):
        _spmm_body(args)

    return pl.kernel(body, out_type=out_type, mesh=mesh,
                     scratch_types=scratch)


def _counts_body(args):
    (cidx_h, dst_h, oh_h,
     cnt_a, cnt_b,
     cidx_v, dst_v, cbuf, zbuf, cnt_sh, csem) = args

    c = lax.axis_index("c")
    s = lax.axis_index("s")
    zero16 = jnp.zeros((16,), jnp.float32)

    def zrow(r, _):
        def zcol(j, _):
            zbuf[r, pl.ds(j * 16, 16)] = zero16
            return 0
        return lax.fori_loop(0, H // 16, zcol, 0)
    lax.fori_loop(0, ZR, zrow, 0)

    def zcopy(k, _):
        pltpu.sync_copy(zbuf, cnt_sh.at[pl.ds(s * RPT + k * ZR, ZR)])
        return 0
    lax.fori_loop(0, RPT // ZR, zcopy, 0)

    pltpu.sync_copy(cidx_h.at[c, s], cidx_v)
    pltpu.sync_copy(dst_h.at[c, s], dst_v)

    plsc.subcore_barrier()

    def chunk(i, _):
        isl = pl.ds(i * K2, K2)
        pltpu.async_copy(oh_h.at[cidx_v.at[isl]], cbuf, csem).wait()
        pltpu.sync_copy(cbuf, cnt_sh.at[dst_v.at[i]], add=True)
        return 0
    lax.fori_loop(0, NCH2, chunk, 0)

    plsc.subcore_barrier()

    wsl = pl.ds(s * RPT, RPT)

    @pl.when(c == 0)
    def _():
        pltpu.sync_copy(cnt_sh.at[wsl], cnt_a.at[wsl])

    @pl.when(c == 1)
    def _():
        pltpu.sync_copy(cnt_sh.at[wsl], cnt_b.at[wsl])


def _make_counts():
    mesh = plsc.VectorSubcoreMesh(core_axis_name="c", subcore_axis_name="s")
    out_type = [jax.ShapeDtypeStruct((NP, H), jnp.float32),
                jax.ShapeDtypeStruct((NP, H), jnp.float32)]
    scratch = [
        pltpu.VMEM((EPT2,), jnp.int32),
        pltpu.VMEM((NCH2, K2), jnp.int32),
        pltpu.VMEM((K2, H), jnp.float32),
        pltpu.VMEM((ZR, H), jnp.float32),
        pltpu.VMEM_SHARED((NP, H), jnp.float32),
        pltpu.SemaphoreType.DMA,
    ]

    def body(*args):
        _counts_body(args)

    return pl.kernel(body, out_type=out_type, mesh=mesh,
                     scratch_types=scratch)


def _pre_call(x, enc_wT, mask_f, memb, alpha):
    def body(x_ref, w_ref, m_ref, e_ref, a_ref, lo_ref, hi_ref):
        xv = x_ref[...]
        av = a_ref[0, 0]
        h = jnp.where(xv >= 0.0, xv, av * xv)
        y = jnp.dot(h, w_ref[...], preferred_element_type=jnp.float32,
                    precision=lax.Precision.HIGHEST)
        m = m_ref[...]
        y = y + m * (e_ref[...] - y)
        lo_ref[...] = y[:, :H]
        hi_ref[...] = y[:, H:]

    return pl.pallas_call(
        body,
        grid=(N // BR,),
        in_specs=[
            pl.BlockSpec((BR, D), lambda i: (i, 0)),
            pl.BlockSpec((D, D), lambda i: (0, 0)),
            pl.BlockSpec((BR, 1), lambda i: (i, 0)),
            pl.BlockSpec((1, D), lambda i: (0, 0)),
            pl.BlockSpec((1, 1), lambda i: (0, 0)),
        ],
        out_specs=[
            pl.BlockSpec((BR, H), lambda i: (i, 0)),
            pl.BlockSpec((BR, H), lambda i: (i, 0)),
        ],
        out_shape=[
            jax.ShapeDtypeStruct((N, H), jnp.float32),
            jax.ShapeDtypeStruct((N, H), jnp.float32),
        ],
    )(x, enc_wT, mask_f, memb, alpha)


def _mlp_call(agg_lo, agg_hi, cnt_a, cnt_b, emb, w1T, b1, w2T, b2):
    def body(lo_ref, hi_ref, ca_ref, cb_ref, e_ref, w1_ref, b1_ref, w2_ref,
             b2_ref, y_ref, ps_ref):
        agg = jnp.concatenate([lo_ref[...], hi_ref[...]], axis=1)
        agg = agg + jnp.dot(ca_ref[...] + cb_ref[...], e_ref[...],
                            preferred_element_type=jnp.float32,
                            precision=lax.Precision.HIGHEST)
        t = jnp.dot(agg, w1_ref[...], preferred_element_type=jnp.float32,
                    precision=lax.Precision.HIGHEST) + b1_ref[...]
        t = jnp.maximum(t, 0.0)
        y = jnp.dot(t, w2_ref[...], preferred_element_type=jnp.float32,
                    precision=lax.Precision.HIGHEST) + b2_ref[...]
        y_ref[...] = y

        @pl.when(pl.program_id(0) == 0)
        def _():
            ps_ref[...] = jnp.zeros_like(ps_ref)
        ps_ref[...] += jnp.concatenate(
            [jnp.sum(y, axis=0, keepdims=True),
             jnp.sum(y * y, axis=0, keepdims=True)], axis=0)

    return pl.pallas_call(
        body,
        grid=(N // BR,),
        in_specs=[
            pl.BlockSpec((BR, H), lambda i: (i, 0)),
            pl.BlockSpec((BR, H), lambda i: (i, 0)),
            pl.BlockSpec((BR, H), lambda i: (i, 0)),
            pl.BlockSpec((BR, H), lambda i: (i, 0)),
            pl.BlockSpec((H, D), lambda i: (0, 0)),
            pl.BlockSpec((D, 2 * D), lambda i: (0, 0)),
            pl.BlockSpec((1, 2 * D), lambda i: (0, 0)),
            pl.BlockSpec((2 * D, D), lambda i: (0, 0)),
            pl.BlockSpec((1, D), lambda i: (0, 0)),
        ],
        out_specs=[
            pl.BlockSpec((BR, D), lambda i: (i, 0)),
            pl.BlockSpec((2, D), lambda i: (0, 0)),
        ],
        out_shape=[
            jax.ShapeDtypeStruct((N, D), jnp.float32),
            jax.ShapeDtypeStruct((2, D), jnp.float32),
        ],
    )(agg_lo, agg_hi, cnt_a, cnt_b, emb, w1T, b1, w2T, b2)


def _bn_call(y, ps, g, be, relu, split):
    def body(y_ref, ps_ref, g_ref, b_ref, *outs):
        mean = ps_ref[0:1, :] * (1.0 / N)
        var = ps_ref[1:2, :] * (1.0 / N) - mean * mean
        inv = g_ref[...] * lax.rsqrt(var + EPS)
        yv = (y_ref[...] - mean) * inv + b_ref[...]
        if relu:
            yv = jnp.maximum(yv, 0.0)
        if split:
            outs[0][...] = yv[:, :H]
            outs[1][...] = yv[:, H:]
        else:
            outs[0][...] = yv

    if split:
        out_specs = [pl.BlockSpec((BR, H), lambda i: (i, 0)),
                     pl.BlockSpec((BR, H), lambda i: (i, 0))]
        out_shape = [jax.ShapeDtypeStruct((N, H), jnp.float32),
                     jax.ShapeDtypeStruct((N, H), jnp.float32)]
    else:
        out_specs = [pl.BlockSpec((BR, D), lambda i: (i, 0))]
        out_shape = [jax.ShapeDtypeStruct((N, D), jnp.float32)]

    return pl.pallas_call(
        body,
        grid=(N // BR,),
        in_specs=[
            pl.BlockSpec((BR, D), lambda i: (i, 0)),
            pl.BlockSpec((2, D), lambda i: (0, 0)),
            pl.BlockSpec((1, D), lambda i: (0, 0)),
            pl.BlockSpec((1, D), lambda i: (0, 0)),
        ],
        out_specs=out_specs,
        out_shape=out_shape,
    )(y, ps, g, be)


def kernel(x, edge_index, edge_attr, masked_tokens, batch, enc_w, prelu_w,
           mask_embed, e1_0, e2_0, w1_0, b1_0, w2_0, b2_0, g_0, be_0,
           e1_1, e2_1, w1_1, b1_1, w2_1, b2_1, g_1, be_1):
    ei = edge_index.astype(jnp.int32)
    ea = edge_attr.astype(jnp.int32)
    src = ei[0].reshape(NT, EPT)
    dst = ei[1].reshape(NT, NCH, K)
    cidx = (ea[:, 0] * 3 + ea[:, 1]).reshape(2, NT, EPT2)
    dst2 = ei[1].reshape(2, NT, NCH2, K2)

    oh_np = np.zeros((16, H), np.float32)
    for a0 in range(3):
        for a1 in range(3):
            oh_np[a0 * 3 + a1, a0] = 1.0
            oh_np[a0 * 3 + a1, 3 + a1] = 1.0
    oh = jnp.asarray(oh_np)

    mask_f = masked_tokens.astype(jnp.float32).reshape(N, 1)
    alpha = prelu_w.reshape(1, 1).astype(jnp.float32)
    memb = mask_embed.reshape(1, D)

    cnt_a, cnt_b = _make_counts()(cidx, dst2, oh)
    cnt_a = cnt_a[:N]
    cnt_b = cnt_b[:N]

    h_lo, h_hi = _pre_call(x, enc_w.T, mask_f, memb, alpha)

    agg_lo, agg_hi = _make_spmm()(h_lo, h_hi, src, dst)
    agg_lo = agg_lo[:N]
    agg_hi = agg_hi[:N]

    emb0 = jnp.concatenate(
        [e1_0[:3], e2_0[:3], jnp.zeros((H - 6, D), jnp.float32)], axis=0)
    y0, ps0 = _mlp_call(agg_lo, agg_hi, cnt_a, cnt_b, emb0, w1_0.T,
                        b1_0.reshape(1, -1), w2_0.T, b2_0.reshape(1, -1))
    h1_lo, h1_hi = _bn_call(y0, ps0, g_0.reshape(1, -1), be_0.reshape(1, -1),
                            relu=True, split=True)

    agg1_lo, agg1_hi = _make_spmm()(h1_lo, h1_hi, src, dst)
    agg1_lo = agg1_lo[:N]
    agg1_hi = agg1_hi[:N]

    emb1 = jnp.concatenate(
        [e1_1[:3], e2_1[:3], jnp.zeros((H - 6, D), jnp.float32)], axis=0)
    y1, ps1 = _mlp_call(agg1_lo, agg1_hi, cnt_a, cnt_b, emb1, w1_1.T,
                        b1_1.reshape(1, -1), w2_1.T, b2_1.reshape(1, -1))
    (out,) = _bn_call(y1, ps1, g_1.reshape(1, -1), be_1.reshape(1, -1),
                      relu=False, split=False)
    return out


# speedup vs baseline: 4.3134x; 1.6711x over previous
import numpy as np
import jax
import jax.numpy as jnp
from jax import lax
from jax.experimental import pallas as pl
from jax.experimental.pallas import tpu as pltpu
from jax.experimental.pallas import tpu_sc as plsc

N = 10000
E = 160000
D = 256
H = 128
NT = 16
EPT = E // NT
K = 80
NCH = EPT // K
NP = 10240
RPT = NP // NT
ZR = 8
BR = 400
EPT2 = E // (2 * NT)
K2 = 40
NCH2 = EPT2 // K2
EPS = 1e-5


def _spmm_body(args):
    (hlo, hhi, src_h, dst_h,
     agglo, agghi,
     src_v, dst_v, buf, zbuf, agg_sh, gsem) = args

    c = lax.axis_index("c")
    s = lax.axis_index("s")
    zero16 = jnp.zeros((16,), jnp.float32)

    def zrow(r, _):
        def zcol(j, _):
            zbuf[r, pl.ds(j * 16, 16)] = zero16
            return 0
        return lax.fori_loop(0, H // 16, zcol, 0)
    lax.fori_loop(0, ZR, zrow, 0)

    def zcopy(k, _):
        pltpu.sync_copy(zbuf, agg_sh.at[pl.ds(s * RPT + k * ZR, ZR)])
        return 0
    lax.fori_loop(0, RPT // ZR, zcopy, 0)

    pltpu.sync_copy(src_h.at[s], src_v)
    pltpu.sync_copy(dst_h.at[s], dst_v)

    plsc.subcore_barrier()

    def chunk(i, _):
        isl = pl.ds(i * K, K)

        @pl.when(c == 0)
        def _():
            pltpu.async_copy(hlo.at[src_v.at[isl]], buf, gsem).wait()

        @pl.when(c == 1)
        def _():
            pltpu.async_copy(hhi.at[src_v.at[isl]], buf, gsem).wait()

        pltpu.sync_copy(buf, agg_sh.at[dst_v.at[i]], add=True)
        return 0
    lax.fori_loop(0, NCH, chunk, 0)

    plsc.subcore_barrier()

    wsl = pl.ds(s * RPT, RPT)

    @pl.when(c == 0)
    def _():
        pltpu.sync_copy(agg_sh.at[wsl], agglo.at[wsl])

    @pl.when(c == 1)
    def _():
        pltpu.sync_copy(agg_sh.at[wsl], agghi.at[wsl])


def _make_spmm():
    mesh = plsc.VectorSubcoreMesh(core_axis_name="c", subcore_axis_name="s")
    out_type = [jax.ShapeDtypeStruct((NP, H), jnp.float32),
                jax.ShapeDtypeStruct((NP, H), jnp.float32)]
    scratch = [
        pltpu.VMEM((EPT,), jnp.int32),
        pltpu.VMEM((NCH, K), jnp.int32),
        pltpu.VMEM((K, H), jnp.float32),
        pltpu.VMEM((ZR, H), jnp.float32),
        pltpu.VMEM_SHARED((NP, H), jnp.float32),
        pltpu.SemaphoreType.DMA,
    ]

    def body(*args):
        _spmm_body(args)

    return pl.kernel(body, out_type=out_type, mesh=mesh,
                     scratch_types=scratch)


def _counts_body(args):
    (cidx_h, dst_h, oh_h,
     cnt_a, cnt_b,
     cidx_v, dst_v, cbuf, zbuf, cnt_sh, csem) = args

    c = lax.axis_index("c")
    s = lax.axis_index("s")
    zero16 = jnp.zeros((16,), jnp.float32)

    def zrow(r, _):
        def zcol(j, _):
            zbuf[r, pl.ds(j * 16, 16)] = zero16
            return 0
        return lax.fori_loop(0, H // 16, zcol, 0)
    lax.fori_loop(0, ZR, zrow, 0)

    def zcopy(k, _):
        pltpu.sync_copy(zbuf, cnt_sh.at[pl.ds(s * RPT + k * ZR, ZR)])
        return 0
    lax.fori_loop(0, RPT // ZR, zcopy, 0)

    pltpu.sync_copy(cidx_h.at[c, s], cidx_v)
    pltpu.sync_copy(dst_h.at[c, s], dst_v)

    plsc.subcore_barrier()

    def chunk(i, _):
        isl = pl.ds(i * K2, K2)
        pltpu.async_copy(oh_h.at[cidx_v.at[isl]], cbuf, csem).wait()
        pltpu.sync_copy(cbuf, cnt_sh.at[dst_v.at[i]], add=True)
        return 0
    lax.fori_loop(0, NCH2, chunk, 0)

    plsc.subcore_barrier()

    wsl = pl.ds(s * RPT, RPT)

    @pl.when(c == 0)
    def _():
        pltpu.sync_copy(cnt_sh.at[wsl], cnt_a.at[wsl])

    @pl.when(c == 1)
    def _():
        pltpu.sync_copy(cnt_sh.at[wsl], cnt_b.at[wsl])


def _make_counts():
    mesh = plsc.VectorSubcoreMesh(core_axis_name="c", subcore_axis_name="s")
    out_type = [jax.ShapeDtypeStruct((NP, H), jnp.float32),
                jax.ShapeDtypeStruct((NP, H), jnp.float32)]
    scratch = [
        pltpu.VMEM((EPT2,), jnp.int32),
        pltpu.VMEM((NCH2, K2), jnp.int32),
        pltpu.VMEM((K2, H), jnp.float32),
        pltpu.VMEM((ZR, H), jnp.float32),
        pltpu.VMEM_SHARED((NP, H), jnp.float32),
        pltpu.SemaphoreType.DMA,
    ]

    def body(*args):
        _counts_body(args)

    return pl.kernel(body, out_type=out_type, mesh=mesh,
                     scratch_types=scratch)


def _pre_call(x, enc_wT, mask_f, memb, alpha):
    def body(x_ref, w_ref, m_ref, e_ref, a_ref, lo_ref, hi_ref):
        xv = x_ref[...]
        av = a_ref[0, 0]
        h = jnp.where(xv >= 0.0, xv, av * xv)
        y = jnp.dot(h, w_ref[...], preferred_element_type=jnp.float32,
                    precision=lax.Precision.HIGHEST)
        m = m_ref[...]
        y = y + m * (e_ref[...] - y)
        lo_ref[...] = y[:, :H]
        hi_ref[...] = y[:, H:]

    return pl.pallas_call(
        body,
        grid=(N // BR,),
        in_specs=[
            pl.BlockSpec((BR, D), lambda i: (i, 0)),
            pl.BlockSpec((D, D), lambda i: (0, 0)),
            pl.BlockSpec((BR, 1), lambda i: (i, 0)),
            pl.BlockSpec((1, D), lambda i: (0, 0)),
            pl.BlockSpec((1, 1), lambda i: (0, 0)),
        ],
        out_specs=[
            pl.BlockSpec((BR, H), lambda i: (i, 0)),
            pl.BlockSpec((BR, H), lambda i: (i, 0)),
        ],
        out_shape=[
            jax.ShapeDtypeStruct((N, H), jnp.float32),
            jax.ShapeDtypeStruct((N, H), jnp.float32),
        ],
    )(x, enc_wT, mask_f, memb, alpha)


def _mlp_call(agg_lo, agg_hi, cnt_a, cnt_b, emb, w1T, b1, w2T, b2):
    def body(lo_ref, hi_ref, ca_ref, cb_ref, e_ref, w1_ref, b1_ref, w2_ref,
             b2_ref, y_ref, ps_ref):
        agg = jnp.concatenate([lo_ref[...], hi_ref[...]], axis=1)
        agg = agg + jnp.dot(ca_ref[...] + cb_ref[...], e_ref[...],
                            preferred_element_type=jnp.float32,
                            precision=lax.Precision.HIGHEST)
        t = jnp.dot(agg, w1_ref[...], preferred_element_type=jnp.float32,
                    precision=lax.Precision.HIGHEST) + b1_ref[...]
        t = jnp.maximum(t, 0.0)
        y = jnp.dot(t, w2_ref[...], preferred_element_type=jnp.float32,
                    precision=lax.Precision.HIGHEST) + b2_ref[...]
        y_ref[...] = y

        @pl.when(pl.program_id(0) == 0)
        def _():
            ps_ref[...] = jnp.zeros_like(ps_ref)
        ps_ref[...] += jnp.concatenate(
            [jnp.sum(y, axis=0, keepdims=True),
             jnp.sum(y * y, axis=0, keepdims=True)], axis=0)

    return pl.pallas_call(
        body,
        grid=(N // BR,),
        in_specs=[
            pl.BlockSpec((BR, H), lambda i: (i, 0)),
            pl.BlockSpec((BR, H), lambda i: (i, 0)),
            pl.BlockSpec((BR, H), lambda i: (i, 0)),
            pl.BlockSpec((BR, H), lambda i: (i, 0)),
            pl.BlockSpec((H, D), lambda i: (0, 0)),
            pl.BlockSpec((D, 2 * D), lambda i: (0, 0)),
            pl.BlockSpec((1, 2 * D), lambda i: (0, 0)),
            pl.BlockSpec((2 * D, D), lambda i: (0, 0)),
            pl.BlockSpec((1, D), lambda i: (0, 0)),
        ],
        out_specs=[
            pl.BlockSpec((BR, D), lambda i: (i, 0)),
            pl.BlockSpec((2, D), lambda i: (0, 0)),
        ],
        out_shape=[
            jax.ShapeDtypeStruct((N, D), jnp.float32),
            jax.ShapeDtypeStruct((2, D), jnp.float32),
        ],
    )(agg_lo, agg_hi, cnt_a, cnt_b, emb, w1T, b1, w2T, b2)


def _bn_call(y, ps, g, be, relu, split):
    def body(y_ref, ps_ref, g_ref, b_ref, *outs):
        mean = ps_ref[0:1, :] * (1.0 / N)
        var = ps_ref[1:2, :] * (1.0 / N) - mean * mean
        inv = g_ref[...] * lax.rsqrt(var + EPS)
        yv = (y_ref[...] - mean) * inv + b_ref[...]
        if relu:
            yv = jnp.maximum(yv, 0.0)
        if split:
            outs[0][...] = yv[:, :H]
            outs[1][...] = yv[:, H:]
        else:
            outs[0][...] = yv

    if split:
        out_specs = [pl.BlockSpec((BR, H), lambda i: (i, 0)),
                     pl.BlockSpec((BR, H), lambda i: (i, 0))]
        out_shape = [jax.ShapeDtypeStruct((N, H), jnp.float32),
                     jax.ShapeDtypeStruct((N, H), jnp.float32)]
    else:
        out_specs = [pl.BlockSpec((BR, D), lambda i: (i, 0))]
        out_shape = [jax.ShapeDtypeStruct((N, D), jnp.float32)]

    return pl.pallas_call(
        body,
        grid=(N // BR,),
        in_specs=[
            pl.BlockSpec((BR, D), lambda i: (i, 0)),
            pl.BlockSpec((2, D), lambda i: (0, 0)),
            pl.BlockSpec((1, D), lambda i: (0, 0)),
            pl.BlockSpec((1, D), lambda i: (0, 0)),
        ],
        out_specs=out_specs,
        out_shape=out_shape,
    )(y, ps, g, be)


def kernel(x, edge_index, edge_attr, masked_tokens, batch, enc_w, prelu_w,
           mask_embed, e1_0, e2_0, w1_0, b1_0, w2_0, b2_0, g_0, be_0,
           e1_1, e2_1, w1_1, b1_1, w2_1, b2_1, g_1, be_1):
    ei = edge_index.astype(jnp.int32)
    ea = edge_attr.astype(jnp.int32)
    src = ei[0].reshape(NT, EPT)
    dst = ei[1].reshape(NT, NCH, K)
    dst2 = ei[1].reshape(2, NT, NCH2, K2)

    oh_np = np.zeros((9, H), np.float32)
    for a0 in range(3):
        for a1 in range(3):
            oh_np[a0 * 3 + a1, a0] = 1.0
            oh_np[a0 * 3 + a1, 3 + a1] = 1.0
    oh = jnp.asarray(np.tile(oh_np, (128, 1)))
    rep = (jnp.arange(E, dtype=jnp.int32) % 128) * 9
    cidx = (ea[:, 0] * 3 + ea[:, 1] + rep).reshape(2, NT, EPT2)

    mask_f = masked_tokens.astype(jnp.float32).reshape(N, 1)
    alpha = prelu_w.reshape(1, 1).astype(jnp.float32)
    memb = mask_embed.reshape(1, D)

    cnt_a, cnt_b = _make_counts()(cidx, dst2, oh)
    cnt_a = cnt_a[:N]
    cnt_b = cnt_b[:N]

    h_lo, h_hi = _pre_call(x, enc_w.T, mask_f, memb, alpha)

    agg_lo, agg_hi = _make_spmm()(h_lo, h_hi, src, dst)
    agg_lo = agg_lo[:N]
    agg_hi = agg_hi[:N]

    emb0 = jnp.concatenate(
        [e1_0[:3], e2_0[:3], jnp.zeros((H - 6, D), jnp.float32)], axis=0)
    y0, ps0 = _mlp_call(agg_lo, agg_hi, cnt_a, cnt_b, emb0, w1_0.T,
                        b1_0.reshape(1, -1), w2_0.T, b2_0.reshape(1, -1))
    h1_lo, h1_hi = _bn_call(y0, ps0, g_0.reshape(1, -1), be_0.reshape(1, -1),
                            relu=True, split=True)

    agg1_lo, agg1_hi = _make_spmm()(h1_lo, h1_hi, src, dst)
    agg1_lo = agg1_lo[:N]
    agg1_hi = agg1_hi[:N]

    emb1 = jnp.concatenate(
        [e1_1[:3], e2_1[:3], jnp.zeros((H - 6, D), jnp.float32)], axis=0)
    y1, ps1 = _mlp_call(agg1_lo, agg1_hi, cnt_a, cnt_b, emb1, w1_1.T,
                        b1_1.reshape(1, -1), w2_1.T, b2_1.reshape(1, -1))
    (out,) = _bn_call(y1, ps1, g_1.reshape(1, -1), be_1.reshape(1, -1),
                      relu=False, split=False)
    return out


# speedup vs baseline: 4.4004x; 1.0202x over previous
import numpy as np
import jax
import jax.numpy as jnp
from jax import lax
from jax.experimental import pallas as pl
from jax.experimental.pallas import tpu as pltpu
from jax.experimental.pallas import tpu_sc as plsc

N = 10000
E = 160000
D = 256
H = 128
NT = 16
EPT = E // NT
K = 80
NCH = EPT // K
NP = 10240
RPT = NP // NT
ZR = 8
BR = 400
EPT2 = E // (2 * NT)
K2 = 128
EPT2P = 5120
NCH2 = EPT2P // K2
EPS = 1e-5


def _spmm_body(args):
    (hlo, hhi, src_h, dst_h,
     agglo, agghi,
     src_v, dst_v, buf, buf1, zbuf, agg_sh, gsem, gsem1) = args

    c = lax.axis_index("c")
    s = lax.axis_index("s")
    zero16 = jnp.zeros((16,), jnp.float32)

    def zrow(r, _):
        def zcol(j, _):
            zbuf[r, pl.ds(j * 16, 16)] = zero16
            return 0
        return lax.fori_loop(0, H // 16, zcol, 0)
    lax.fori_loop(0, ZR, zrow, 0)

    def zcopy(k, _):
        pltpu.sync_copy(zbuf, agg_sh.at[pl.ds(s * RPT + k * ZR, ZR)])
        return 0
    lax.fori_loop(0, RPT // ZR, zcopy, 0)

    pltpu.sync_copy(src_h.at[s], src_v)
    pltpu.sync_copy(dst_h.at[s], dst_v)

    plsc.subcore_barrier()

    def pair(j, _):
        i0 = 2 * j
        i1 = i0 + 1
        sl0 = pl.ds(i0 * K, K)
        sl1 = pl.ds(i1 * K, K)

        @pl.when(c == 0)
        def _():
            cp0 = pltpu.async_copy(hlo.at[src_v.at[sl0]], buf, gsem)
            cp1 = pltpu.async_copy(hlo.at[src_v.at[sl1]], buf1, gsem1)
            cp0.wait()
            pltpu.sync_copy(buf, agg_sh.at[dst_v.at[i0]], add=True)
            cp1.wait()
            pltpu.sync_copy(buf1, agg_sh.at[dst_v.at[i1]], add=True)

        @pl.when(c == 1)
        def _():
            cp0 = pltpu.async_copy(hhi.at[src_v.at[sl0]], buf, gsem)
            cp1 = pltpu.async_copy(hhi.at[src_v.at[sl1]], buf1, gsem1)
            cp0.wait()
            pltpu.sync_copy(buf, agg_sh.at[dst_v.at[i0]], add=True)
            cp1.wait()
            pltpu.sync_copy(buf1, agg_sh.at[dst_v.at[i1]], add=True)
        return 0
    lax.fori_loop(0, NCH // 2, pair, 0)

    ilast = NCH - 1
    sll = pl.ds(ilast * K, K)

    @pl.when(c == 0)
    def _():
        pltpu.async_copy(hlo.at[src_v.at[sll]], buf, gsem).wait()

    @pl.when(c == 1)
    def _():
        pltpu.async_copy(hhi.at[src_v.at[sll]], buf, gsem).wait()

    pltpu.sync_copy(buf, agg_sh.at[dst_v.at[ilast]], add=True)

    plsc.subcore_barrier()

    wsl = pl.ds(s * RPT, RPT)

    @pl.when(c == 0)
    def _():
        pltpu.sync_copy(agg_sh.at[wsl], agglo.at[wsl])

    @pl.when(c == 1)
    def _():
        pltpu.sync_copy(agg_sh.at[wsl], agghi.at[wsl])


def _make_spmm():
    mesh = plsc.VectorSubcoreMesh(core_axis_name="c", subcore_axis_name="s")
    out_type = [jax.ShapeDtypeStruct((NP, H), jnp.float32),
                jax.ShapeDtypeStruct((NP, H), jnp.float32)]
    scratch = [
        pltpu.VMEM((EPT,), jnp.int32),
        pltpu.VMEM((NCH, K), jnp.int32),
        pltpu.VMEM((K, H), jnp.float32),
        pltpu.VMEM((K, H), jnp.float32),
        pltpu.VMEM((ZR, H), jnp.float32),
        pltpu.VMEM_SHARED((NP, H), jnp.float32),
        pltpu.SemaphoreType.DMA,
        pltpu.SemaphoreType.DMA,
    ]

    def body(*args):
        _spmm_body(args)

    return pl.kernel(body, out_type=out_type, mesh=mesh,
                     scratch_types=scratch)


def _counts_body(args):
    (cidx_h, dst_h, oh_h,
     cnt_a, cnt_b,
     cidx_v, dst_v, cbuf, cbuf1, zbuf, cnt_sh, csem, csem1) = args

    c = lax.axis_index("c")
    s = lax.axis_index("s")
    zero16 = jnp.zeros((16,), jnp.float32)

    def zrow(r, _):
        def zcol(j, _):
            zbuf[r, pl.ds(j * 16, 16)] = zero16
            return 0
        return lax.fori_loop(0, H // 16, zcol, 0)
    lax.fori_loop(0, ZR, zrow, 0)

    def zcopy(k, _):
        pltpu.sync_copy(zbuf, cnt_sh.at[pl.ds(s * RPT + k * ZR, ZR)])
        return 0
    lax.fori_loop(0, RPT // ZR, zcopy, 0)

    pltpu.sync_copy(cidx_h.at[c, s], cidx_v)
    pltpu.sync_copy(dst_h.at[c, s], dst_v)

    plsc.subcore_barrier()

    def pair(j, _):
        i0 = 2 * j
        i1 = i0 + 1
        cp0 = pltpu.async_copy(oh_h.at[cidx_v.at[pl.ds(i0 * K2, K2)]],
                               cbuf, csem)
        cp1 = pltpu.async_copy(oh_h.at[cidx_v.at[pl.ds(i1 * K2, K2)]],
                               cbuf1, csem1)
        cp0.wait()
        pltpu.sync_copy(cbuf, cnt_sh.at[dst_v.at[i0]], add=True)
        cp1.wait()
        pltpu.sync_copy(cbuf1, cnt_sh.at[dst_v.at[i1]], add=True)
        return 0
    lax.fori_loop(0, NCH2 // 2, pair, 0)

    plsc.subcore_barrier()

    wsl = pl.ds(s * RPT, RPT)

    @pl.when(c == 0)
    def _():
        pltpu.sync_copy(cnt_sh.at[wsl], cnt_a.at[wsl])

    @pl.when(c == 1)
    def _():
        pltpu.sync_copy(cnt_sh.at[wsl], cnt_b.at[wsl])


def _make_counts():
    mesh = plsc.VectorSubcoreMesh(core_axis_name="c", subcore_axis_name="s")
    out_type = [jax.ShapeDtypeStruct((NP, H), jnp.float32),
                jax.ShapeDtypeStruct((NP, H), jnp.float32)]
    scratch = [
        pltpu.VMEM((EPT2P,), jnp.int32),
        pltpu.VMEM((NCH2, K2), jnp.int32),
        pltpu.VMEM((K2, H), jnp.float32),
        pltpu.VMEM((K2, H), jnp.float32),
        pltpu.VMEM((ZR, H), jnp.float32),
        pltpu.VMEM_SHARED((NP, H), jnp.float32),
        pltpu.SemaphoreType.DMA,
        pltpu.SemaphoreType.DMA,
    ]

    def body(*args):
        _counts_body(args)

    return pl.kernel(body, out_type=out_type, mesh=mesh,
                     scratch_types=scratch)


def _pre_call(x, enc_wT, mask_f, memb, alpha):
    def body(x_ref, w_ref, m_ref, e_ref, a_ref, lo_ref, hi_ref):
        xv = x_ref[...]
        av = a_ref[0, 0]
        h = jnp.where(xv >= 0.0, xv, av * xv)
        y = jnp.dot(h, w_ref[...], preferred_element_type=jnp.float32,
                    precision=lax.Precision.HIGHEST)
        m = m_ref[...]
        y = y + m * (e_ref[...] - y)
        lo_ref[...] = y[:, :H]
        hi_ref[...] = y[:, H:]

    return pl.pallas_call(
        body,
        grid=(N // BR,),
        in_specs=[
            pl.BlockSpec((BR, D), lambda i: (i, 0)),
            pl.BlockSpec((D, D), lambda i: (0, 0)),
            pl.BlockSpec((BR, 1), lambda i: (i, 0)),
            pl.BlockSpec((1, D), lambda i: (0, 0)),
            pl.BlockSpec((1, 1), lambda i: (0, 0)),
        ],
        out_specs=[
            pl.BlockSpec((BR, H), lambda i: (i, 0)),
            pl.BlockSpec((BR, H), lambda i: (i, 0)),
        ],
        out_shape=[
            jax.ShapeDtypeStruct((N, H), jnp.float32),
            jax.ShapeDtypeStruct((N, H), jnp.float32),
        ],
    )(x, enc_wT, mask_f, memb, alpha)


def _mlp_call(agg_lo, agg_hi, cnt_a, cnt_b, emb, w1T, b1, w2T, b2):
    def body(lo_ref, hi_ref, ca_ref, cb_ref, e_ref, w1_ref, b1_ref, w2_ref,
             b2_ref, y_ref, ps_ref):
        agg = jnp.concatenate([lo_ref[...], hi_ref[...]], axis=1)
        agg = agg + jnp.dot(ca_ref[...] + cb_ref[...], e_ref[...],
                            preferred_element_type=jnp.float32,
                            precision=lax.Precision.HIGHEST)
        t = jnp.dot(agg, w1_ref[...], preferred_element_type=jnp.float32,
                    precision=lax.Precision.HIGHEST) + b1_ref[...]
        t = jnp.maximum(t, 0.0)
        y = jnp.dot(t, w2_ref[...], preferred_element_type=jnp.float32,
                    precision=lax.Precision.HIGHEST) + b2_ref[...]
        y_ref[...] = y

        @pl.when(pl.program_id(0) == 0)
        def _():
            ps_ref[...] = jnp.zeros_like(ps_ref)
        ps_ref[...] += jnp.concatenate(
            [jnp.sum(y, axis=0, keepdims=True),
             jnp.sum(y * y, axis=0, keepdims=True)], axis=0)

    return pl.pallas_call(
        body,
        grid=(N // BR,),
        in_specs=[
            pl.BlockSpec((BR, H), lambda i: (i, 0)),
            pl.BlockSpec((BR, H), lambda i: (i, 0)),
            pl.BlockSpec((BR, H), lambda i: (i, 0)),
            pl.BlockSpec((BR, H), lambda i: (i, 0)),
            pl.BlockSpec((H, D), lambda i: (0, 0)),
            pl.BlockSpec((D, 2 * D), lambda i: (0, 0)),
            pl.BlockSpec((1, 2 * D), lambda i: (0, 0)),
            pl.BlockSpec((2 * D, D), lambda i: (0, 0)),
            pl.BlockSpec((1, D), lambda i: (0, 0)),
        ],
        out_specs=[
            pl.BlockSpec((BR, D), lambda i: (i, 0)),
            pl.BlockSpec((2, D), lambda i: (0, 0)),
        ],
        out_shape=[
            jax.ShapeDtypeStruct((N, D), jnp.float32),
            jax.ShapeDtypeStruct((2, D), jnp.float32),
        ],
    )(agg_lo, agg_hi, cnt_a, cnt_b, emb, w1T, b1, w2T, b2)


def _bn_call(y, ps, g, be, relu, split):
    def body(y_ref, ps_ref, g_ref, b_ref, *outs):
        mean = ps_ref[0:1, :] * (1.0 / N)
        var = ps_ref[1:2, :] * (1.0 / N) - mean * mean
        inv = g_ref[...] * lax.rsqrt(var + EPS)
        yv = (y_ref[...] - mean) * inv + b_ref[...]
        if relu:
            yv = jnp.maximum(yv, 0.0)
        if split:
            outs[0][...] = yv[:, :H]
            outs[1][...] = yv[:, H:]
        else:
            outs[0][...] = yv

    if split:
        out_specs = [pl.BlockSpec((BR, H), lambda i: (i, 0)),
                     pl.BlockSpec((BR, H), lambda i: (i, 0))]
        out_shape = [jax.ShapeDtypeStruct((N, H), jnp.float32),
                     jax.ShapeDtypeStruct((N, H), jnp.float32)]
    else:
        out_specs = [pl.BlockSpec((BR, D), lambda i: (i, 0))]
        out_shape = [jax.ShapeDtypeStruct((N, D), jnp.float32)]

    return pl.pallas_call(
        body,
        grid=(N // BR,),
        in_specs=[
            pl.BlockSpec((BR, D), lambda i: (i, 0)),
            pl.BlockSpec((2, D), lambda i: (0, 0)),
            pl.BlockSpec((1, D), lambda i: (0, 0)),
            pl.BlockSpec((1, D), lambda i: (0, 0)),
        ],
        out_specs=out_specs,
        out_shape=out_shape,
    )(y, ps, g, be)


def kernel(x, edge_index, edge_attr, masked_tokens, batch, enc_w, prelu_w,
           mask_embed, e1_0, e2_0, w1_0, b1_0, w2_0, b2_0, g_0, be_0,
           e1_1, e2_1, w1_1, b1_1, w2_1, b2_1, g_1, be_1):
    ei = edge_index.astype(jnp.int32)
    ea = edge_attr.astype(jnp.int32)
    src = ei[0].reshape(NT, EPT)
    dst = ei[1].reshape(NT, NCH, K)
    dst2 = jnp.pad(ei[1].reshape(2, NT, EPT2),
                   ((0, 0), (0, 0), (0, EPT2P - EPT2)),
                   constant_values=NP - 1).reshape(2, NT, NCH2, K2)

    oh_np = np.zeros((9, H), np.float32)
    for a0 in range(3):
        for a1 in range(3):
            oh_np[a0 * 3 + a1, a0] = 1.0
            oh_np[a0 * 3 + a1, 3 + a1] = 1.0
    oh = jnp.asarray(np.tile(oh_np, (128, 1)))
    rep = (jnp.arange(E, dtype=jnp.int32) % 128) * 9
    cidx = jnp.pad((ea[:, 0] * 3 + ea[:, 1] + rep).reshape(2, NT, EPT2),
                   ((0, 0), (0, 0), (0, EPT2P - EPT2)))

    mask_f = masked_tokens.astype(jnp.float32).reshape(N, 1)
    alpha = prelu_w.reshape(1, 1).astype(jnp.float32)
    memb = mask_embed.reshape(1, D)

    cnt_a, cnt_b = _make_counts()(cidx, dst2, oh)
    cnt_a = cnt_a[:N]
    cnt_b = cnt_b[:N]

    h_lo, h_hi = _pre_call(x, enc_w.T, mask_f, memb, alpha)

    agg_lo, agg_hi = _make_spmm()(h_lo, h_hi, src, dst)
    agg_lo = agg_lo[:N]
    agg_hi = agg_hi[:N]

    emb0 = jnp.concatenate(
        [e1_0[:3], e2_0[:3], jnp.zeros((H - 6, D), jnp.float32)], axis=0)
    y0, ps0 = _mlp_call(agg_lo, agg_hi, cnt_a, cnt_b, emb0, w1_0.T,
                        b1_0.reshape(1, -1), w2_0.T, b2_0.reshape(1, -1))
    h1_lo, h1_hi = _bn_call(y0, ps0, g_0.reshape(1, -1), be_0.reshape(1, -1),
                            relu=True, split=True)

    agg1_lo, agg1_hi = _make_spmm()(h1_lo, h1_hi, src, dst)
    agg1_lo = agg1_lo[:N]
    agg1_hi = agg1_hi[:N]

    emb1 = jnp.concatenate(
        [e1_1[:3], e2_1[:3], jnp.zeros((H - 6, D), jnp.float32)], axis=0)
    y1, ps1 = _mlp_call(agg1_lo, agg1_hi, cnt_a, cnt_b, emb1, w1_1.T,
                        b1_1.reshape(1, -1), w2_1.T, b2_1.reshape(1, -1))
    (out,) = _bn_call(y1, ps1, g_1.reshape(1, -1), be_1.reshape(1, -1),
                      relu=False, split=False)
    return out
